# baseline (speedup 1.0000x reference)
import functools

import jax
import jax.numpy as jnp
from jax import lax
from jax.experimental import pallas as pl
from jax.experimental.pallas import tpu as pltpu
from jax.experimental.pallas import tpu_sc as plsc

NUSR = 5000
NITM = 5000
N = NUSR + NITM
D = 128
NNZ = 320000
BATCH = 4096
NC = 2
NS = 16
NT = NC * NS
NPAD = NT * 320
HALF = NPAD // 2
CSPAN = NPAD // NS
NNZP = NT * 10240
EPTS = NNZP // NT
F32 = jnp.float32
I32 = jnp.int32

_mesh = plsc.VectorSubcoreMesh(core_axis_name="c", subcore_axis_name="s")

_dnums = lax.GatherDimensionNumbers(
    offset_dims=(), collapsed_slice_dims=(0,), start_index_map=(0,)
)


def _perm(v, idxs):
    return lax.gather(
        v, idxs[:, None], _dnums, (1,),
        mode=lax.GatherScatterMode.PROMISE_IN_BOUNDS,
    )


def _lane():
    return lax.broadcasted_iota(I32, (16,), 0)


def _splat_lane(v, l):
    return _perm(v, jnp.full((16,), l, I32))


def _excl_prefix(v):
    lane = _lane()
    incl = v
    for k in (1, 2, 4, 8):
        shifted = _perm(incl, jnp.maximum(lane - k, 0))
        incl = incl + jnp.where(lane >= k, shifted, 0)
    return incl - v, incl


def _cdiv640(cv):
    return ((cv >> 7) * 6554) >> 15


def _bucket(rv, cv):
    hi = 16 + (((rv - HALF) >> 31) * 16)
    return hi + _cdiv640(cv)


@functools.partial(
    pl.kernel,
    out_type=jax.ShapeDtypeStruct((NT * 32,), I32),
    mesh=_mesh,
    scratch_types=[
        pltpu.VMEM((1024,), I32),
        pltpu.VMEM((1024,), I32),
        pltpu.VMEM((32,), I32),
    ],
)
def _s1(row_h, col_h, cnt_out, row_v, col_v, hist_v):
    c = lax.axis_index("c")
    s = lax.axis_index("s")
    w = s * NC + c
    base = w * EPTS
    lane = _lane()
    one = jnp.full((16,), 1, I32)
    zero = jnp.zeros((16,), I32)

    hist_v[pl.ds(0, 16)] = zero
    hist_v[pl.ds(16, 16)] = zero

    def _chunk(k, carry):
        pltpu.sync_copy(row_h.at[pl.ds(base + k * 1024, 1024)], row_v)
        pltpu.sync_copy(col_h.at[pl.ds(base + k * 1024, 1024)], col_v)

        def _vec(g, carry2):
            rv = row_v[pl.ds(g * 16, 16)]
            cv = col_v[pl.ds(g * 16, 16)]
            b = _bucket(rv, cv)
            a0 = hist_v[pl.ds(0, 16)]
            a1 = hist_v[pl.ds(16, 16)]
            for l in range(16):
                bl = _splat_lane(b, l)
                a0 = a0 + jnp.where(lane == bl, one, zero)
                a1 = a1 + jnp.where(lane == bl - 16, one, zero)
            hist_v[pl.ds(0, 16)] = a0
            hist_v[pl.ds(16, 16)] = a1
            return carry2

        return lax.fori_loop(0, 64, _vec, carry)

    lax.fori_loop(0, EPTS // 1024, _chunk, 0)
    pltpu.sync_copy(hist_v, cnt_out.at[pl.ds(w * 32, 32)])


@functools.partial(
    pl.kernel,
    out_type=(
        jax.ShapeDtypeStruct((NNZP,), I32),
        jax.ShapeDtypeStruct((NNZP,), I32),
        jax.ShapeDtypeStruct((NNZP,), F32),
    ),
    mesh=_mesh,
    scratch_types=[
        pltpu.VMEM((1024,), I32),
        pltpu.VMEM((1024,), I32),
        pltpu.VMEM((1024,), I32),
        pltpu.VMEM((1024,), F32),
        pltpu.VMEM((8, 128), I32),
        pltpu.SemaphoreType.DMA,
    ],
)
def _s2(row_h, col_h, val_h, cnt_h, lrow_out, lcol_out, val_out,
        cnt_v, row_v, col_v, val_v, pos_v, sem):
    c = lax.axis_index("c")
    s = lax.axis_index("s")
    w = s * NC + c
    base = w * EPTS
    lane = _lane()
    one = jnp.full((16,), 1, I32)
    zero = jnp.zeros((16,), I32)

    pltpu.sync_copy(cnt_h, cnt_v)
    tot0 = zero
    tot1 = zero
    arr0 = zero
    arr1 = zero
    for wp in range(NT):
        cv0 = cnt_v[pl.ds(wp * 32, 16)]
        cv1 = cnt_v[pl.ds(wp * 32 + 16, 16)]
        tot0 = tot0 + cv0
        tot1 = tot1 + cv1
        bm = jnp.where(jnp.full((16,), wp, I32) < jnp.full((16,), w, I32), one, zero)
        arr0 = arr0 + cv0 * bm
        arr1 = arr1 + cv1 * bm
    base0, incl0 = _excl_prefix(tot0)
    base1, _ = _excl_prefix(tot1)
    sum0 = _splat_lane(incl0, 15)
    cur0 = base0 + arr0
    cur1 = base1 + sum0 + arr1

    def _chunk(k, carry):
        cur0, cur1 = carry
        off = base + k * 1024
        pltpu.sync_copy(row_h.at[pl.ds(off, 1024)], row_v)
        pltpu.sync_copy(col_h.at[pl.ds(off, 1024)], col_v)
        pltpu.sync_copy(val_h.at[pl.ds(off, 1024)], val_v)

        def _vec(g, carry2):
            cur0, cur1 = carry2
            rv = row_v[pl.ds(g * 16, 16)]
            cv = col_v[pl.ds(g * 16, 16)]
            b = _bucket(rv, cv)
            row_v[pl.ds(g * 16, 16)] = rv - jnp.where(rv >= HALF, HALF, 0)
            col_v[pl.ds(g * 16, 16)] = cv - _cdiv640(cv) * CSPAN
            pos = zero
            for l in range(16):
                bl = _splat_lane(b, l)
                lo = ((bl - 16) >> 31) * (-1)
                p_lo = _perm(cur0, jnp.minimum(bl, 15))
                p_hi = _perm(cur1, jnp.maximum(bl - 16, 0))
                pick = lo * p_lo + (1 - lo) * p_hi
                pos = jnp.where(lane == l, pick, pos)
                cur0 = cur0 + jnp.where(lane == bl, lo, zero)
                cur1 = cur1 + jnp.where(lane == bl - 16, 1 - lo, zero)
            r = g // 8
            j = g % 8
            pos_v[r, pl.ds(j * 16, 16)] = pos
            return (cur0, cur1)

        cur0, cur1 = lax.fori_loop(0, 64, _vec, (cur0, cur1))
        descs = []
        for r in range(8):
            sl = pl.ds(r * 128, 128)
            descs.append(
                pltpu.async_copy(row_v.at[sl], lrow_out.at[pos_v.at[r]], sem)
            )
            descs.append(
                pltpu.async_copy(col_v.at[sl], lcol_out.at[pos_v.at[r]], sem)
            )
            descs.append(
                pltpu.async_copy(val_v.at[sl], val_out.at[pos_v.at[r]], sem)
            )
        for dsc in descs:
            dsc.wait()
        return (cur0, cur1)

    lax.fori_loop(0, EPTS // 1024, _chunk, (cur0, cur1))


QTR = HALF // 2


@functools.partial(
    pl.kernel,
    out_type=jax.ShapeDtypeStruct((NPAD, D), F32),
    mesh=_mesh,
    scratch_types=[
        pltpu.VMEM((CSPAN, D), F32),
        pltpu.VMEM((1024,), I32),
        pltpu.VMEM((8, 128), I32),
        pltpu.VMEM((1024,), I32),
        pltpu.VMEM((1024,), F32),
        pltpu.VMEM((128, D), F32),
        pltpu.VMEM((56, D), F32),
        pltpu.VMEM_SHARED((QTR, D), F32),
        pltpu.SMEM((8,), I32),
        pltpu.SMEM((16,), I32),
        pltpu.SemaphoreType.DMA,
    ],
)
def _layer(ego, lrow2, lcol, val, cnt_h, out,
           slab, cnt_v, lrow_v, lcol_v, val_v, stage, zer_v, acc,
           se_sm, src_sm, sem):
    c = lax.axis_index("c")
    s = lax.axis_index("s")
    b_mine = c * 16 + s
    lane = _lane()

    zv = jnp.zeros((16,), F32)

    def _zb(r, carry):
        for jj in range(8):
            zer_v[r, pl.ds(16 * jj, 16)] = zv
        return carry

    lax.fori_loop(0, 56, _zb, 0, unroll=4)

    pltpu.sync_copy(ego.at[pl.ds(s * CSPAN, CSPAN)], slab)

    pltpu.sync_copy(cnt_h, cnt_v)
    zero = jnp.zeros((16,), I32)
    tot0 = zero
    tot1 = zero
    for wp in range(NT):
        tot0 = tot0 + cnt_v[pl.ds(wp * 32, 16)]
        tot1 = tot1 + cnt_v[pl.ds(wp * 32 + 16, 16)]
    base0, incl0 = _excl_prefix(tot0)
    base1, _ = _excl_prefix(tot1)
    sum0 = _splat_lane(incl0, 15)
    bl_mine = jnp.full((16,), b_mine, I32)
    lo_mask = jnp.where(bl_mine < 16, 1, 0)
    start_v = (
        lo_mask * _perm(base0, jnp.minimum(bl_mine, 15))
        + (1 - lo_mask) * _perm(base1 + sum0, jnp.maximum(bl_mine - 16, 0))
    )
    cnt_mine = (
        lo_mask * _perm(tot0, jnp.minimum(bl_mine, 15))
        + (1 - lo_mask) * _perm(tot1, jnp.maximum(bl_mine - 16, 0))
    )
    start_nv = jnp.where(lane == 0, start_v, zero)
    end_nv = jnp.where(lane == 0, start_v + cnt_mine, zero)
    se_sm[0] = start_nv[0]
    se_sm[1] = end_nv[0]
    start = se_sm[0]
    end = se_sm[1]
    k0 = start >> 10
    k1 = (end + 1023) >> 10

    for q in range(2):
        pltpu.sync_copy(zer_v, acc.at[pl.ds(s * 160, 56)])
        pltpu.sync_copy(zer_v, acc.at[pl.ds(s * 160 + 56, 56)])
        pltpu.sync_copy(zer_v.at[pl.ds(0, 48)], acc.at[pl.ds(s * 160 + 112, 48)])
        plsc.subcore_barrier()

        def _chunk(k, carry, q=q):
            off = k * 1024
            pltpu.sync_copy(lrow2.at[pl.ds(k * 8, 8)], lrow_v)
            pltpu.sync_copy(lcol.at[pl.ds(off, 1024)], lcol_v)
            pltpu.sync_copy(val.at[pl.ds(off, 1024)], val_v)
            for hh in range(8):
                def _vec(g, carry2, hh=hh, q=q):
                    gi = hh * 8 + g
                    pos = jnp.full((16,), 0, I32) + off + gi * 16 + lane
                    vlv = val_v[pl.ds(gi * 16, 16)]
                    sl16 = pl.ds(g * 16, 16)
                    lrv = lrow_v[hh, sl16]
                    lq = lrv - q * QTR
                    keep = (pos >= start) & (pos < end)
                    inq = (lq >= 0) & (lq < QTR)
                    vlv = jnp.where(keep & inq, vlv, 0.0)
                    lrow_v[hh, sl16] = jnp.clip(lq, 0, QTR - 1)
                    lcv = lcol_v[pl.ds(gi * 16, 16)]
                    for l in range(16):
                        src_sm[l] = lcv[l]
                    for l in range(16):
                        e = g * 16 + l
                        vv = jnp.full((16,), vlv[l], F32)
                        src = src_sm[l]
                        for jj in range(8):
                            sl = pl.ds(16 * jj, 16)
                            stage[e, sl] = slab[src, sl] * vv
                    return carry2

                lax.fori_loop(0, 8, _vec, 0)
                pltpu.sync_copy(stage, acc.at[lrow_v.at[hh]], add=True)
            return carry

        lax.fori_loop(k0, k1, _chunk, 0)
        plsc.subcore_barrier()
        pltpu.sync_copy(
            acc.at[pl.ds(s * 160, 160)],
            out.at[pl.ds(c * HALF + q * QTR + s * 160, 160)],
        )
        plsc.subcore_barrier()


BPT = BATCH // NT


@functools.partial(
    pl.kernel,
    out_type=(
        jax.ShapeDtypeStruct((BATCH,), F32),
        jax.ShapeDtypeStruct((BATCH,), F32),
    ),
    mesh=_mesh,
    scratch_types=[
        pltpu.VMEM((NT, BPT), I32),
        pltpu.VMEM((NT, BPT), I32),
        pltpu.VMEM((NT, BPT), I32),
        pltpu.VMEM((BPT, D), F32),
        pltpu.VMEM((BPT, D), F32),
        pltpu.VMEM((BPT, D), F32),
        pltpu.VMEM((BPT,), F32),
        pltpu.VMEM((BPT,), F32),
        pltpu.SemaphoreType.DMA,
    ],
)
def _score(ego, u2, i2, j2, out_p, out_n,
           idx_u, idx_i, idx_j, ur, ir, jr, sp, sn, sem):
    c = lax.axis_index("c")
    s = lax.axis_index("s")
    wid = s * NC + c
    pltpu.sync_copy(u2, idx_u)
    pltpu.sync_copy(i2, idx_i)
    pltpu.sync_copy(j2, idx_j)
    descs = [
        pltpu.async_copy(ego.at[idx_u.at[wid]], ur, sem),
        pltpu.async_copy(ego.at[idx_i.at[wid]], ir, sem),
        pltpu.async_copy(ego.at[idx_j.at[wid]], jr, sem),
    ]
    for dsc in descs:
        dsc.wait()

    lane = _lane()

    def _allsum(v):
        for k in (8, 4, 2, 1):
            v = v + _perm(v, jnp.bitwise_xor(lane, k))
        return v

    def _dot(g, carry):
        pv = jnp.zeros((16,), F32)
        nv = jnp.zeros((16,), F32)
        for l in range(16):
            k = g * 16 + l
            accp = jnp.zeros((16,), F32)
            accn = jnp.zeros((16,), F32)
            for jj in range(8):
                sl = pl.ds(16 * jj, 16)
                uv = ur[k, sl]
                accp = accp + uv * ir[k, sl]
                accn = accn + uv * jr[k, sl]
            pv = jnp.where(lane == l, _allsum(accp), pv)
            nv = jnp.where(lane == l, _allsum(accn), nv)
        sp[pl.ds(g * 16, 16)] = pv
        sn[pl.ds(g * 16, 16)] = nv
        return carry

    lax.fori_loop(0, BPT // 16, _dot, 0)
    pltpu.sync_copy(sp, out_p.at[pl.ds(wid * BPT, BPT)])
    pltpu.sync_copy(sn, out_n.at[pl.ds(wid * BPT, BPT)])


def _loss_body(p_ref, n_ref, o_ref):
    p = p_ref[...]
    n = n_ref[...]
    lp = jnp.maximum(p, 0.0) - p + jnp.log1p(jnp.exp(-jnp.abs(p)))
    ln = jnp.maximum(n, 0.0) + jnp.log1p(jnp.exp(-jnp.abs(n)))
    total = (jnp.sum(lp) + jnp.sum(ln)) * (0.5 / BATCH)
    o_ref[...] = jnp.reshape(total, (1, 1))


def _loss(sp, sn):
    return pl.pallas_call(
        _loss_body,
        out_shape=jax.ShapeDtypeStruct((1, 1), F32),
    )(sp, sn)


def kernel(user_embedding, item_embedding, adj_values, adj_indices, u, i, j):
    ego = jnp.concatenate(
        [user_embedding, item_embedding, jnp.zeros((NPAD - N, D), F32)], axis=0
    )
    pad = NNZP - NNZ
    k = jnp.arange(pad, dtype=I32)
    row_p = jnp.concatenate([adj_indices[0].astype(I32), (k % 2) * HALF])
    col_p = jnp.concatenate([adj_indices[1].astype(I32), (k % 16) * CSPAN])
    val_p = jnp.concatenate([adj_values.astype(F32), jnp.zeros((pad,), F32)])
    u2 = u.astype(I32).reshape(NT, BPT)
    i2 = (i.astype(I32) + NUSR).reshape(NT, BPT)
    j2 = (j.astype(I32) + NUSR).reshape(NT, BPT)

    counts = _s1(row_p, col_p)
    lrow_s, lcol_s, val_s = _s2(row_p, col_p, val_p, counts)
    lrow2 = lrow_s.reshape(NNZP // 128, 128)

    for _ in range(3):
        ego = _layer(ego, lrow2, lcol_s, val_s, counts)
    sp, sn = _score(ego, u2, i2, j2)
    loss = _loss(sp.reshape(NT, BPT), sn.reshape(NT, BPT))
    return loss[0, 0]


# speedup vs baseline: 1.4389x; 1.4389x over previous
import functools

import jax
import jax.numpy as jnp
from jax import lax
from jax.experimental import pallas as pl
from jax.experimental.pallas import tpu as pltpu
from jax.experimental.pallas import tpu_sc as plsc

NUSR = 5000
NITM = 5000
N = NUSR + NITM
D = 128
NNZ = 320000
BATCH = 4096
NC = 2
NS = 16
NT = NC * NS
NPAD = NT * 320
IDXW = 128
NNZP = NS * 160 * IDXW
EPT16 = NNZP // NS
F32 = jnp.float32

HALF = NPAD // 2
QTR = HALF // 2
ACCR = QTR + 16

_mesh = plsc.VectorSubcoreMesh(core_axis_name="c", subcore_axis_name="s")


@functools.partial(
    pl.kernel,
    out_type=jax.ShapeDtypeStruct((NPAD, D), F32),
    mesh=_mesh,
    scratch_types=[
        pltpu.VMEM((8, IDXW), jnp.int32),
        pltpu.VMEM((8, IDXW), jnp.int32),
        pltpu.VMEM((1024,), F32),
        pltpu.VMEM((IDXW, D), F32),
        pltpu.VMEM((32, D), F32),
        pltpu.VMEM_SHARED((NPAD, D), F32),
        pltpu.VMEM_SHARED((ACCR, D), F32),
        pltpu.SemaphoreType.DMA,
    ],
)
def _layer(ego, col2, row2, val, out, idx_c, idx_r, val_v, rows_v, zer_v,
           ego_sh, acc, sem):
    c = lax.axis_index("c")
    s = lax.axis_index("s")

    zv = jnp.zeros((16,), F32)

    def _zb(r, carry):
        for jj in range(8):
            zer_v[r, pl.ds(16 * jj, 16)] = zv
        return carry

    lax.fori_loop(0, 32, _zb, 0, unroll=4)

    pltpu.sync_copy(ego.at[pl.ds(s * 640, 640)], ego_sh.at[pl.ds(s * 640, 640)])

    erow0 = s * (EPT16 // IDXW)
    vbase = s * EPT16

    for q in range(2):
        rbase = c * HALF + q * QTR
        for i in range(5):
            pltpu.sync_copy(zer_v, acc.at[pl.ds(s * 160 + i * 32, 32)])

        @pl.when(s == 0)
        def _zt():
            pltpu.sync_copy(zer_v.at[pl.ds(0, 16)], acc.at[pl.ds(QTR, 16)])

        plsc.subcore_barrier()

        def _blk(blk, carry, q=q, rbase=rbase):
            r0 = erow0 + blk * 8
            pltpu.sync_copy(col2.at[pl.ds(r0, 8)], idx_c)
            pltpu.sync_copy(row2.at[pl.ds(r0, 8)], idx_r)
            pltpu.sync_copy(val.at[pl.ds(vbase + blk * 1024, 1024)], val_v)
            for r in range(8):
                for jj in range(8):
                    sl = pl.ds(16 * jj, 16)
                    rv = idx_r[r, sl] - rbase
                    ok = (rv >= 0) & (rv < QTR)
                    idx_r[r, sl] = jnp.where(ok, rv, QTR)
            for r in range(8):
                pltpu.async_copy(ego_sh.at[idx_c.at[r]], rows_v, sem).wait()

                def _scale(g, carry2, r=r):
                    vlv = val_v[pl.ds(r * 128 + g * 16, 16)]
                    for l in range(16):
                        vv = jnp.full((16,), vlv[l], F32)
                        e = g * 16 + l
                        for jj in range(8):
                            sl = pl.ds(16 * jj, 16)
                            rows_v[e, sl] = rows_v[e, sl] * vv
                    return carry2

                lax.fori_loop(0, 8, _scale, 0)
                pltpu.sync_copy(rows_v, acc.at[idx_r.at[r]], add=True)
            return carry

        lax.fori_loop(0, EPT16 // 1024, _blk, 0)
        plsc.subcore_barrier()
        pltpu.sync_copy(
            acc.at[pl.ds(s * 160, 160)], out.at[pl.ds(rbase + s * 160, 160)]
        )
        plsc.subcore_barrier()


BPT = BATCH // NT


@functools.partial(
    pl.kernel,
    out_type=(
        jax.ShapeDtypeStruct((BATCH,), F32),
        jax.ShapeDtypeStruct((BATCH,), F32),
    ),
    mesh=_mesh,
    scratch_types=[
        pltpu.VMEM((NT, BPT), jnp.int32),
        pltpu.VMEM((NT, BPT), jnp.int32),
        pltpu.VMEM((NT, BPT), jnp.int32),
        pltpu.VMEM((BPT, D), F32),
        pltpu.VMEM((BPT, D), F32),
        pltpu.VMEM((BPT, D), F32),
        pltpu.VMEM((BPT,), F32),
        pltpu.VMEM((BPT,), F32),
        pltpu.SemaphoreType.DMA,
    ],
)
def _score(ego, u2, i2, j2, out_p, out_n,
           idx_u, idx_i, idx_j, ur, ir, jr, sp, sn, sem):
    c = lax.axis_index("c")
    s = lax.axis_index("s")
    wid = s * NC + c
    pltpu.sync_copy(u2, idx_u)
    pltpu.sync_copy(i2, idx_i)
    pltpu.sync_copy(j2, idx_j)
    descs = [
        pltpu.async_copy(ego.at[idx_u.at[wid]], ur, sem),
        pltpu.async_copy(ego.at[idx_i.at[wid]], ir, sem),
        pltpu.async_copy(ego.at[idx_j.at[wid]], jr, sem),
    ]
    for dsc in descs:
        dsc.wait()

    lane = lax.broadcasted_iota(jnp.int32, (16,), 0)
    dnums = lax.GatherDimensionNumbers(
        offset_dims=(), collapsed_slice_dims=(0,), start_index_map=(0,)
    )

    def _perm(v, idxs):
        return lax.gather(
            v, idxs[:, None], dnums, (1,),
            mode=lax.GatherScatterMode.PROMISE_IN_BOUNDS,
        )

    def _allsum(v):
        for k in (8, 4, 2, 1):
            v = v + _perm(v, jnp.bitwise_xor(lane, k))
        return v

    def _dot(g, carry):
        pv = jnp.zeros((16,), F32)
        nv = jnp.zeros((16,), F32)
        for l in range(16):
            k = g * 16 + l
            accp = jnp.zeros((16,), F32)
            accn = jnp.zeros((16,), F32)
            for jj in range(8):
                sl = pl.ds(16 * jj, 16)
                uv = ur[k, sl]
                accp = accp + uv * ir[k, sl]
                accn = accn + uv * jr[k, sl]
            pv = jnp.where(lane == l, _allsum(accp), pv)
            nv = jnp.where(lane == l, _allsum(accn), nv)
        sp[pl.ds(g * 16, 16)] = pv
        sn[pl.ds(g * 16, 16)] = nv
        return carry

    lax.fori_loop(0, BPT // 16, _dot, 0)
    pltpu.sync_copy(sp, out_p.at[pl.ds(wid * BPT, BPT)])
    pltpu.sync_copy(sn, out_n.at[pl.ds(wid * BPT, BPT)])


def _loss_body(p_ref, n_ref, o_ref):
    p = p_ref[...]
    n = n_ref[...]
    lp = jnp.maximum(p, 0.0) - p + jnp.log1p(jnp.exp(-jnp.abs(p)))
    ln = jnp.maximum(n, 0.0) + jnp.log1p(jnp.exp(-jnp.abs(n)))
    total = (jnp.sum(lp) + jnp.sum(ln)) * (0.5 / BATCH)
    o_ref[...] = jnp.reshape(total, (1, 1))


def _loss(sp, sn):
    return pl.pallas_call(
        _loss_body,
        out_shape=jax.ShapeDtypeStruct((1, 1), F32),
    )(sp, sn)


def kernel(user_embedding, item_embedding, adj_values, adj_indices, u, i, j):
    ego = jnp.concatenate(
        [user_embedding, item_embedding, jnp.zeros((NPAD - N, D), F32)], axis=0
    )
    pad = NNZP - NNZ
    row2 = jnp.concatenate(
        [adj_indices[0].astype(jnp.int32), jnp.zeros((pad,), jnp.int32)]
    ).reshape(NNZP // IDXW, IDXW)
    col2 = jnp.concatenate(
        [adj_indices[1].astype(jnp.int32), jnp.zeros((pad,), jnp.int32)]
    ).reshape(NNZP // IDXW, IDXW)
    val = jnp.concatenate([adj_values.astype(F32), jnp.zeros((pad,), F32)])
    u2 = u.astype(jnp.int32).reshape(NT, BPT)
    i2 = (i.astype(jnp.int32) + NUSR).reshape(NT, BPT)
    j2 = (j.astype(jnp.int32) + NUSR).reshape(NT, BPT)

    for _ in range(3):
        ego = _layer(ego, col2, row2, val)
    sp, sn = _score(ego, u2, i2, j2)
    loss = _loss(sp.reshape(NT, BPT), sn.reshape(NT, BPT))
    return loss[0, 0]


# speedup vs baseline: 2.7727x; 1.9269x over previous
import functools

import jax
import jax.numpy as jnp
from jax import lax
from jax.experimental import pallas as pl
from jax.experimental.pallas import tpu as pltpu
from jax.experimental.pallas import tpu_sc as plsc

NUSR = 5000
NITM = 5000
N = NUSR + NITM
D = 128
NNZ = 320000
BATCH = 4096
NC = 2
NS = 16
NT = NC * NS
NPAD = NT * 320
IDXW = 128
NNZP = NS * 160 * IDXW
EPT16 = NNZP // NS
F32 = jnp.float32

EPT = NNZP // NT

_mesh = plsc.VectorSubcoreMesh(core_axis_name="c", subcore_axis_name="s")


@functools.partial(
    pl.kernel,
    out_type=jax.ShapeDtypeStruct((NC, NPAD, D), F32),
    mesh=_mesh,
    scratch_types=[
        pltpu.VMEM((8, IDXW), jnp.int32),
        pltpu.VMEM((8, IDXW), jnp.int32),
        pltpu.VMEM((1024,), F32),
        pltpu.VMEM((IDXW, D), F32),
        pltpu.VMEM((IDXW, D), F32),
        pltpu.VMEM((32, D), F32),
        pltpu.VMEM_SHARED((NPAD, D), F32),
        pltpu.SemaphoreType.DMA,
    ],
)
def _layer(ego, col2, row2, val, out, idx_c, idx_r, val_v, buf_a, buf_b,
           zer_v, acc, sem):
    c = lax.axis_index("c")
    s = lax.axis_index("s")
    wid = s * NC + c

    zv = jnp.zeros((16,), F32)

    def _zb(r, carry):
        for jj in range(8):
            zer_v[r, pl.ds(16 * jj, 16)] = zv
        return carry

    lax.fori_loop(0, 32, _zb, 0, unroll=4)
    for i in range(20):
        pltpu.sync_copy(zer_v, acc.at[pl.ds(s * 640 + i * 32, 32)])
    plsc.subcore_barrier()

    erow0 = wid * (EPT // IDXW)
    vbase = wid * EPT
    bufs = (buf_a, buf_b)

    def _blk(blk, carry):
        r0 = erow0 + blk * 8
        pltpu.sync_copy(col2.at[pl.ds(r0, 8)], idx_c)
        pltpu.sync_copy(row2.at[pl.ds(r0, 8)], idx_r)
        pltpu.sync_copy(val.at[pl.ds(vbase + blk * 1024, 1024)], val_v)
        descs = [None] * 8
        descs[0] = pltpu.async_copy(ego.at[idx_c.at[0]], bufs[0], sem)
        descs[1] = pltpu.async_copy(ego.at[idx_c.at[1]], bufs[1], sem)
        for r in range(8):
            descs[r].wait()
            buf = bufs[r % 2]

            def _scale(g, carry2, r=r, buf=buf):
                vlv = val_v[pl.ds(r * 128 + g * 16, 16)]
                for l in range(16):
                    vv = jnp.full((16,), vlv[l], F32)
                    e = g * 16 + l
                    for jj in range(8):
                        sl = pl.ds(16 * jj, 16)
                        buf[e, sl] = buf[e, sl] * vv
                return carry2

            lax.fori_loop(0, 8, _scale, 0)
            pltpu.sync_copy(buf, acc.at[idx_r.at[r]], add=True)
            if r + 2 < 8:
                descs[r + 2] = pltpu.async_copy(
                    ego.at[idx_c.at[r + 2]], bufs[r % 2], sem
                )
        return carry

    lax.fori_loop(0, EPT // 1024, _blk, 0)
    plsc.subcore_barrier()
    pltpu.sync_copy(
        acc.at[pl.ds(s * 640, 640)], out.at[c].at[pl.ds(s * 640, 640)]
    )


def _combine_body(p_ref, o_ref):
    o_ref[...] = p_ref[0] + p_ref[1]


def _combine(p):
    return pl.pallas_call(
        _combine_body,
        out_shape=jax.ShapeDtypeStruct((NPAD, D), F32),
        grid=(4,),
        in_specs=[pl.BlockSpec((2, NPAD // 4, D), lambda i: (0, i, 0))],
        out_specs=pl.BlockSpec((NPAD // 4, D), lambda i: (i, 0)),
    )(p)


BPT = BATCH // NT


@functools.partial(
    pl.kernel,
    out_type=(
        jax.ShapeDtypeStruct((BATCH,), F32),
        jax.ShapeDtypeStruct((BATCH,), F32),
    ),
    mesh=_mesh,
    scratch_types=[
        pltpu.VMEM((NT, BPT), jnp.int32),
        pltpu.VMEM((NT, BPT), jnp.int32),
        pltpu.VMEM((NT, BPT), jnp.int32),
        pltpu.VMEM((BPT, D), F32),
        pltpu.VMEM((BPT, D), F32),
        pltpu.VMEM((BPT, D), F32),
        pltpu.VMEM((BPT,), F32),
        pltpu.VMEM((BPT,), F32),
        pltpu.SemaphoreType.DMA,
    ],
)
def _score(ego, u2, i2, j2, out_p, out_n,
           idx_u, idx_i, idx_j, ur, ir, jr, sp, sn, sem):
    c = lax.axis_index("c")
    s = lax.axis_index("s")
    wid = s * NC + c
    pltpu.sync_copy(u2, idx_u)
    pltpu.sync_copy(i2, idx_i)
    pltpu.sync_copy(j2, idx_j)
    descs = [
        pltpu.async_copy(ego.at[idx_u.at[wid]], ur, sem),
        pltpu.async_copy(ego.at[idx_i.at[wid]], ir, sem),
        pltpu.async_copy(ego.at[idx_j.at[wid]], jr, sem),
    ]
    for dsc in descs:
        dsc.wait()

    lane = lax.broadcasted_iota(jnp.int32, (16,), 0)
    dnums = lax.GatherDimensionNumbers(
        offset_dims=(), collapsed_slice_dims=(0,), start_index_map=(0,)
    )

    def _perm(v, idxs):
        return lax.gather(
            v, idxs[:, None], dnums, (1,),
            mode=lax.GatherScatterMode.PROMISE_IN_BOUNDS,
        )

    def _allsum(v):
        for k in (8, 4, 2, 1):
            v = v + _perm(v, jnp.bitwise_xor(lane, k))
        return v

    def _dot(g, carry):
        pv = jnp.zeros((16,), F32)
        nv = jnp.zeros((16,), F32)
        for l in range(16):
            k = g * 16 + l
            accp = jnp.zeros((16,), F32)
            accn = jnp.zeros((16,), F32)
            for jj in range(8):
                sl = pl.ds(16 * jj, 16)
                uv = ur[k, sl]
                accp = accp + uv * ir[k, sl]
                accn = accn + uv * jr[k, sl]
            pv = jnp.where(lane == l, _allsum(accp), pv)
            nv = jnp.where(lane == l, _allsum(accn), nv)
        sp[pl.ds(g * 16, 16)] = pv
        sn[pl.ds(g * 16, 16)] = nv
        return carry

    lax.fori_loop(0, BPT // 16, _dot, 0)
    pltpu.sync_copy(sp, out_p.at[pl.ds(wid * BPT, BPT)])
    pltpu.sync_copy(sn, out_n.at[pl.ds(wid * BPT, BPT)])


def _loss_body(p_ref, n_ref, o_ref):
    p = p_ref[...]
    n = n_ref[...]
    lp = jnp.maximum(p, 0.0) - p + jnp.log1p(jnp.exp(-jnp.abs(p)))
    ln = jnp.maximum(n, 0.0) + jnp.log1p(jnp.exp(-jnp.abs(n)))
    total = (jnp.sum(lp) + jnp.sum(ln)) * (0.5 / BATCH)
    o_ref[...] = jnp.reshape(total, (1, 1))


def _loss(sp, sn):
    return pl.pallas_call(
        _loss_body,
        out_shape=jax.ShapeDtypeStruct((1, 1), F32),
    )(sp, sn)


def kernel(user_embedding, item_embedding, adj_values, adj_indices, u, i, j):
    ego = jnp.concatenate(
        [user_embedding, item_embedding, jnp.zeros((NPAD - N, D), F32)], axis=0
    )
    pad = NNZP - NNZ
    row2 = jnp.concatenate(
        [adj_indices[0].astype(jnp.int32), jnp.zeros((pad,), jnp.int32)]
    ).reshape(NNZP // IDXW, IDXW)
    col2 = jnp.concatenate(
        [adj_indices[1].astype(jnp.int32), jnp.zeros((pad,), jnp.int32)]
    ).reshape(NNZP // IDXW, IDXW)
    val = jnp.concatenate([adj_values.astype(F32), jnp.zeros((pad,), F32)])
    u2 = u.astype(jnp.int32).reshape(NT, BPT)
    i2 = (i.astype(jnp.int32) + NUSR).reshape(NT, BPT)
    j2 = (j.astype(jnp.int32) + NUSR).reshape(NT, BPT)

    for _ in range(3):
        ego = _combine(_layer(ego, col2, row2, val))
    sp, sn = _score(ego, u2, i2, j2)
    loss = _loss(sp.reshape(NT, BPT), sn.reshape(NT, BPT))
    return loss[0, 0]
